# baseline (speedup 1.0000x reference)
import functools

import jax
import jax.numpy as jnp
from jax import lax
from jax.experimental import pallas as pl
from jax.experimental.pallas import tpu as pltpu
from jax.experimental.pallas import tpu_sc as plsc

_L = 8192
_K = 256
_H = 16
_LANES = 16

_NC = 2
_NS = 16
_NW = _NC * _NS
_ROWS_PER_W = _L // _NW
_ROW_CHUNK = 64


def _sc_dist_body(coords_flat_hbm, idx_hbm, out_hbm, tab_x, tab_y, tab_z,
                  idx_buf, dist_buf, sem_in0, sem_in1, sem_out0, sem_out1):
    cid = lax.axis_index("c")
    sid = lax.axis_index("s")
    wid = sid * _NC + cid
    row0 = wid * _ROWS_PER_W

    pltpu.sync_copy(coords_flat_hbm.at[pl.ds(0, _L)], tab_x)
    pltpu.sync_copy(coords_flat_hbm.at[pl.ds(_L, _L)], tab_y)
    pltpu.sync_copy(coords_flat_hbm.at[pl.ds(2 * _L, _L)], tab_z)

    sem_in = (sem_in0, sem_in1)
    sem_out = (sem_out0, sem_out1)
    n_bufs = _ROWS_PER_W // _ROW_CHUNK

    def start_in(c):
        return pltpu.async_copy(
            idx_hbm.at[pl.ds(row0 + c * _ROW_CHUNK, _ROW_CHUNK)],
            idx_buf.at[c % 2], sem_in[c % 2])

    in_h = [None] * n_bufs
    out_h = [None] * n_bufs
    in_h[0] = start_in(0)
    for c in range(n_bufs):
        cur = c % 2
        if c + 1 < n_bufs:
            in_h[c + 1] = start_in(c + 1)
        in_h[c].wait()
        if c >= 2:
            out_h[c - 2].wait()
        base = row0 + c * _ROW_CHUNK

        @plsc.parallel_loop(0, _ROW_CHUNK, unroll=4)
        def row_body(r, cur=cur, base=base):
            rowv = jnp.full((_LANES,), base + r, jnp.int32)
            qx = plsc.load_gather(tab_x, [rowv])
            qy = plsc.load_gather(tab_y, [rowv])
            qz = plsc.load_gather(tab_z, [rowv])
            for v in range(_K // _LANES):
                idxv = idx_buf[cur, r, pl.ds(v * _LANES, _LANES)]
                gx = plsc.load_gather(tab_x, [idxv])
                gy = plsc.load_gather(tab_y, [idxv])
                gz = plsc.load_gather(tab_z, [idxv])
                d = jnp.abs(qx - gx) + jnp.abs(qy - gy) + jnp.abs(qz - gz)
                dist_buf[cur, r, pl.ds(v * _LANES, _LANES)] = d

        out_h[c] = pltpu.async_copy(
            dist_buf.at[cur], out_hbm.at[pl.ds(base, _ROW_CHUNK)],
            sem_out[cur])
    out_h[n_bufs - 2].wait()
    out_h[n_bufs - 1].wait()


_sc_dist = pl.kernel(
    _sc_dist_body,
    out_type=jax.ShapeDtypeStruct((_L, _K), jnp.float32),
    mesh=plsc.VectorSubcoreMesh(core_axis_name="c", subcore_axis_name="s"),
    compiler_params=pltpu.CompilerParams(needs_layout_passes=False),
    scratch_types=[
        pltpu.VMEM((_L,), jnp.float32),
        pltpu.VMEM((_L,), jnp.float32),
        pltpu.VMEM((_L,), jnp.float32),
        pltpu.VMEM((2, _ROW_CHUNK, _K), jnp.int32),
        pltpu.VMEM((2, _ROW_CHUNK, _K), jnp.float32),
        pltpu.SemaphoreType.DMA,
        pltpu.SemaphoreType.DMA,
        pltpu.SemaphoreType.DMA,
        pltpu.SemaphoreType.DMA,
    ],
)

_BLK = 1024


def _tc_scale_body(dist_ref, decay_ref, out_ref):
    out_ref[...] = dist_ref[...][None, :, :] * decay_ref[...]


_tc_scale = pl.pallas_call(
    _tc_scale_body,
    grid=(_L // _BLK,),
    in_specs=[
        pl.BlockSpec((_BLK, _K), lambda i: (i, 0)),
        pl.BlockSpec((_H, 1, 1), lambda i: (0, 0, 0)),
    ],
    out_specs=pl.BlockSpec((_H, _BLK, _K), lambda i: (0, i, 0)),
    out_shape=jax.ShapeDtypeStruct((_H, _L, _K), jnp.float32),
)


def kernel(all_coords, idx_tensor, decay):
    dist = _sc_dist(all_coords.T.reshape(-1), idx_tensor)
    return _tc_scale(dist, decay.reshape(_H, 1, 1))


# speedup vs baseline: 1.0069x; 1.0069x over previous
import functools

import jax
import jax.numpy as jnp
from jax import lax
from jax.experimental import pallas as pl
from jax.experimental.pallas import tpu as pltpu
from jax.experimental.pallas import tpu_sc as plsc

_L = 8192
_K = 256
_H = 16
_LANES = 16

_NC = 2
_NS = 16
_NW = _NC * _NS
_ROWS_PER_W = _L // _NW
_ROW_CHUNK = 64


def _sc_dist_body(coords_flat_hbm, idx_hbm, out_hbm, tab_x, tab_y, tab_z,
                  idx_buf, dist_buf, sem_in0, sem_in1, sem_out0, sem_out1):
    cid = lax.axis_index("c")
    sid = lax.axis_index("s")
    wid = sid * _NC + cid
    row0 = wid * _ROWS_PER_W

    pltpu.sync_copy(coords_flat_hbm.at[pl.ds(0, _L)], tab_x)
    pltpu.sync_copy(coords_flat_hbm.at[pl.ds(_L, _L)], tab_y)
    pltpu.sync_copy(coords_flat_hbm.at[pl.ds(2 * _L, _L)], tab_z)

    sem_in = (sem_in0, sem_in1)
    sem_out = (sem_out0, sem_out1)
    n_bufs = _ROWS_PER_W // _ROW_CHUNK

    def start_in(c):
        return pltpu.async_copy(
            idx_hbm.at[pl.ds(row0 + c * _ROW_CHUNK, _ROW_CHUNK)],
            idx_buf.at[c % 2], sem_in[c % 2])

    in_h = [None] * n_bufs
    out_h = [None] * n_bufs
    in_h[0] = start_in(0)
    for c in range(n_bufs):
        cur = c % 2
        if c + 1 < n_bufs:
            in_h[c + 1] = start_in(c + 1)
        in_h[c].wait()
        if c >= 2:
            out_h[c - 2].wait()
        base = row0 + c * _ROW_CHUNK

        @plsc.parallel_loop(0, _ROW_CHUNK, unroll=4)
        def row_body(r, cur=cur, base=base):
            rowv = jnp.full((_LANES,), base + r, jnp.int32)
            qx = plsc.load_gather(tab_x, [rowv])
            qy = plsc.load_gather(tab_y, [rowv])
            qz = plsc.load_gather(tab_z, [rowv])
            for v in range(_K // _LANES):
                idxv = idx_buf[cur, r, pl.ds(v * _LANES, _LANES)]
                gx = plsc.load_gather(tab_x, [idxv])
                gy = plsc.load_gather(tab_y, [idxv])
                gz = plsc.load_gather(tab_z, [idxv])
                d = jnp.abs(qx - gx) + jnp.abs(qy - gy) + jnp.abs(qz - gz)
                dist_buf[cur, r, pl.ds(v * _LANES, _LANES)] = d

        out_h[c] = pltpu.async_copy(
            dist_buf.at[cur], out_hbm.at[pl.ds(base, _ROW_CHUNK)],
            sem_out[cur])
    out_h[n_bufs - 2].wait()
    out_h[n_bufs - 1].wait()


_sc_dist = pl.kernel(
    _sc_dist_body,
    out_type=jax.ShapeDtypeStruct((_L, _K), jnp.float32),
    mesh=plsc.VectorSubcoreMesh(core_axis_name="c", subcore_axis_name="s"),
    compiler_params=pltpu.CompilerParams(needs_layout_passes=False),
    scratch_types=[
        pltpu.VMEM((_L,), jnp.float32),
        pltpu.VMEM((_L,), jnp.float32),
        pltpu.VMEM((_L,), jnp.float32),
        pltpu.VMEM((2, _ROW_CHUNK, _K), jnp.int32),
        pltpu.VMEM((2, _ROW_CHUNK, _K), jnp.float32),
        pltpu.SemaphoreType.DMA,
        pltpu.SemaphoreType.DMA,
        pltpu.SemaphoreType.DMA,
        pltpu.SemaphoreType.DMA,
    ],
)

_BLK = 512


def _tc_scale_body(dist_ref, decay_ref, out_ref):
    out_ref[...] = dist_ref[...][None, :, :] * decay_ref[...]


_tc_scale = pl.pallas_call(
    _tc_scale_body,
    grid=(_L // _BLK,),
    in_specs=[
        pl.BlockSpec((_BLK, _K), lambda i: (i, 0)),
        pl.BlockSpec((_H, 1, 1), lambda i: (0, 0, 0)),
    ],
    out_specs=pl.BlockSpec((_H, _BLK, _K), lambda i: (0, i, 0)),
    out_shape=jax.ShapeDtypeStruct((_H, _L, _K), jnp.float32),
)


def kernel(all_coords, idx_tensor, decay):
    dist = _sc_dist(all_coords.T.reshape(-1), idx_tensor)
    return _tc_scale(dist, decay.reshape(_H, 1, 1))


# speedup vs baseline: 1.0827x; 1.0753x over previous
import functools

import jax
import jax.numpy as jnp
from jax import lax
from jax.experimental import pallas as pl
from jax.experimental.pallas import tpu as pltpu
from jax.experimental.pallas import tpu_sc as plsc

_L = 8192
_K = 256
_H = 16
_LANES = 16

_NC = 2
_NS = 16
_NW = _NC * _NS
_ROWS_PER_W = _L // _NW
_ROW_CHUNK = 64


def _sc_dist_body(coords_flat_hbm, idx_hbm, out_hbm, tab_x, tab_y, tab_z,
                  idx_buf, dist_buf, sem_in0, sem_in1, sem_out0, sem_out1):
    cid = lax.axis_index("c")
    sid = lax.axis_index("s")
    wid = sid * _NC + cid
    row0 = wid * _ROWS_PER_W

    pltpu.sync_copy(coords_flat_hbm.at[pl.ds(0, _L)], tab_x)
    pltpu.sync_copy(coords_flat_hbm.at[pl.ds(_L, _L)], tab_y)
    pltpu.sync_copy(coords_flat_hbm.at[pl.ds(2 * _L, _L)], tab_z)

    sem_in = (sem_in0, sem_in1)
    sem_out = (sem_out0, sem_out1)
    n_bufs = _ROWS_PER_W // _ROW_CHUNK

    def start_in(c):
        return pltpu.async_copy(
            idx_hbm.at[pl.ds(row0 + c * _ROW_CHUNK, _ROW_CHUNK)],
            idx_buf.at[c % 2], sem_in[c % 2])

    in_h = [None] * n_bufs
    out_h = [None] * n_bufs
    in_h[0] = start_in(0)
    for c in range(n_bufs):
        cur = c % 2
        if c + 1 < n_bufs:
            in_h[c + 1] = start_in(c + 1)
        in_h[c].wait()
        if c >= 2:
            out_h[c - 2].wait()
        base = row0 + c * _ROW_CHUNK

        @plsc.parallel_loop(0, _ROW_CHUNK)
        def row_body(r, cur=cur, base=base):
            rowv = jnp.full((_LANES,), base + r, jnp.int32)
            qx = plsc.load_gather(tab_x, [rowv])
            qy = plsc.load_gather(tab_y, [rowv])
            qz = plsc.load_gather(tab_z, [rowv])
            for v in range(_K // _LANES):
                idxv = idx_buf[cur, r, pl.ds(v * _LANES, _LANES)]
                gx = plsc.load_gather(tab_x, [idxv])
                gy = plsc.load_gather(tab_y, [idxv])
                gz = plsc.load_gather(tab_z, [idxv])
                d = jnp.abs(qx - gx) + jnp.abs(qy - gy) + jnp.abs(qz - gz)
                dist_buf[cur, r, pl.ds(v * _LANES, _LANES)] = d

        out_h[c] = pltpu.async_copy(
            dist_buf.at[cur], out_hbm.at[pl.ds(base, _ROW_CHUNK)],
            sem_out[cur])
    out_h[n_bufs - 2].wait()
    out_h[n_bufs - 1].wait()


_sc_dist = pl.kernel(
    _sc_dist_body,
    out_type=jax.ShapeDtypeStruct((_L, _K), jnp.float32),
    mesh=plsc.VectorSubcoreMesh(core_axis_name="c", subcore_axis_name="s"),
    compiler_params=pltpu.CompilerParams(needs_layout_passes=False),
    scratch_types=[
        pltpu.VMEM((_L,), jnp.float32),
        pltpu.VMEM((_L,), jnp.float32),
        pltpu.VMEM((_L,), jnp.float32),
        pltpu.VMEM((2, _ROW_CHUNK, _K), jnp.int32),
        pltpu.VMEM((2, _ROW_CHUNK, _K), jnp.float32),
        pltpu.SemaphoreType.DMA,
        pltpu.SemaphoreType.DMA,
        pltpu.SemaphoreType.DMA,
        pltpu.SemaphoreType.DMA,
    ],
)

_BLK = 512


def _tc_scale_body(dist_ref, decay_ref, out_ref):
    out_ref[...] = dist_ref[...][None, :, :] * decay_ref[...]


_tc_scale = pl.pallas_call(
    _tc_scale_body,
    grid=(_L // _BLK,),
    in_specs=[
        pl.BlockSpec((_BLK, _K), lambda i: (i, 0)),
        pl.BlockSpec((_H, 1, 1), lambda i: (0, 0, 0)),
    ],
    out_specs=pl.BlockSpec((_H, _BLK, _K), lambda i: (0, i, 0)),
    out_shape=jax.ShapeDtypeStruct((_H, _L, _K), jnp.float32),
)


def kernel(all_coords, idx_tensor, decay):
    dist = _sc_dist(all_coords.T.reshape(-1), idx_tensor)
    return _tc_scale(dist, decay.reshape(_H, 1, 1))
